# baseline (speedup 1.0000x reference)
import functools

import jax
import jax.numpy as jnp
from jax.experimental import pallas as pl
from jax.experimental.pallas import tpu as pltpu

_NX = 2
_NW = 4


def _gcn_body(x_hbm, w_hbm, b_ref, o_ref, xv, wv, sems, *, d_feat):
    n = xv.shape[0]
    out_dim = wv.shape[0]
    xs = n // _NX
    ws = out_dim // _NW

    x_copies = []
    for k in range(_NX):
        c = pltpu.make_async_copy(
            x_hbm.at[pl.ds(k * xs, xs), :], xv.at[pl.ds(k * xs, xs), :],
            sems.at[k])
        c.start()
        x_copies.append(c)
    w_copies = []
    for k in range(_NW):
        c = pltpu.make_async_copy(
            w_hbm.at[pl.ds(k * ws, ws), :], wv.at[pl.ds(k * ws, ws), :],
            sems.at[_NX + k])
        c.start()
        w_copies.append(c)

    for c in x_copies:
        c.wait()
    h = jnp.sum(xv[...], axis=1, keepdims=True) * (1.0 / d_feat)

    for c in w_copies:
        c.wait()
    o_ref[...] = b_ref[...] + jnp.dot(wv[...], h,
                                      preferred_element_type=jnp.float32)


def kernel(x, edge_index, W, b):
    del edge_index
    n, d = x.shape
    out_dim = W.shape[0]

    body = functools.partial(_gcn_body, d_feat=d)
    out = pl.pallas_call(
        body,
        in_specs=[
            pl.BlockSpec(memory_space=pl.ANY),
            pl.BlockSpec(memory_space=pl.ANY),
            pl.BlockSpec((out_dim, 1), lambda: (0, 0)),
        ],
        out_specs=pl.BlockSpec((out_dim, 1), lambda: (0, 0)),
        out_shape=jax.ShapeDtypeStruct((out_dim, 1), jnp.float32),
        scratch_shapes=[
            pltpu.VMEM((n, d), jnp.float32),
            pltpu.VMEM((out_dim, n), jnp.float32),
            pltpu.SemaphoreType.DMA((_NX + _NW,)),
        ],
    )(x, W, b.reshape(out_dim, 1))
    return out.reshape(out_dim)
